# baseline (speedup 1.0000x reference)
import jax
import jax.numpy as jnp
from jax.experimental import pallas as pl
from jax.experimental.pallas import tpu as pltpu

N = 4096
IN_F = 64
OUT_F = 64
BR = 512
NBUF = 3
NUM_BLK = N // BR


def _gcn_kernel(inp_ref, w_ref, b_ref, adj_hbm, out_ref, support_ref, buf_ref, sem):
    support_ref[...] = jnp.dot(
        inp_ref[...], w_ref[...], preferred_element_type=jnp.float32
    ).astype(jnp.bfloat16)

    def copy(k, slot):
        return pltpu.make_async_copy(
            adj_hbm.at[pl.ds(k * BR, BR), :],
            buf_ref.at[slot],
            sem.at[slot],
        )

    for s in range(min(NBUF, NUM_BLK)):
        copy(s, s).start()

    for k in range(NUM_BLK):
        slot = k % NBUF
        copy(k, slot).wait()
        t = jnp.dot(
            buf_ref[slot].astype(jnp.bfloat16),
            support_ref[...],
            preferred_element_type=jnp.float32,
        )
        out_ref[pl.ds(k * BR, BR), :] = t + b_ref[...]
        nk = k + NBUF
        if nk < NUM_BLK:
            copy(nk, slot).start()


def kernel(input, adj, W, b):
    b2 = b.reshape(1, OUT_F)
    return pl.pallas_call(
        _gcn_kernel,
        in_specs=[
            pl.BlockSpec(memory_space=pltpu.MemorySpace.VMEM),
            pl.BlockSpec(memory_space=pltpu.MemorySpace.VMEM),
            pl.BlockSpec(memory_space=pltpu.MemorySpace.VMEM),
            pl.BlockSpec(memory_space=pltpu.MemorySpace.HBM),
        ],
        out_specs=pl.BlockSpec(memory_space=pltpu.MemorySpace.VMEM),
        out_shape=jax.ShapeDtypeStruct((N, OUT_F), jnp.float32),
        scratch_shapes=[
            pltpu.VMEM((N, OUT_F), jnp.bfloat16),
            pltpu.VMEM((NBUF, BR, N), jnp.float32),
            pltpu.SemaphoreType.DMA((NBUF,)),
        ],
    )(input, W, b2, adj)
